# baseline (device time: 14853 ns/iter reference)
import jax
import jax.numpy as jnp
from jax import lax
from jax.experimental import pallas as pl
from jax.experimental.pallas import tpu as pltpu

N_DEV = 8


def kernel(x):
    m, n = x.shape[-2], x.shape[-1]
    x = x.reshape(m, n)
    rows = m // N_DEV

    def body(x_ref, out_ref, rs_buf, red_ref,
             rs_send, rs_recv, ag_send, ag_recv):
        my_x = lax.axis_index("x")
        my_y = lax.axis_index("y")
        my_z = lax.axis_index("z")
        my_lid = my_x * 4 + my_y * 2 + my_z

        def peer(r):
            rx, ry, rz = (r >> 2) & 1, (r >> 1) & 1, r & 1
            px = (1 - my_x) if rx else my_x
            py = (1 - my_y) if ry else my_y
            pz = (1 - my_z) if rz else my_z
            return (px, py, pz), px * 4 + py * 2 + pz

        peers = [peer(r) for r in range(1, N_DEV)]

        barrier_sem = pltpu.get_barrier_semaphore()
        for tgt, _ in peers:
            pl.semaphore_signal(
                barrier_sem, inc=1,
                device_id=tgt, device_id_type=pl.DeviceIdType.MESH,
            )
        pl.semaphore_wait(barrier_sem, N_DEV - 1)

        rs = []
        for i, (tgt, tgt_lid) in enumerate(peers):
            d = pltpu.make_async_remote_copy(
                src_ref=x_ref.at[pl.ds(tgt_lid * rows, rows)],
                dst_ref=rs_buf.at[i],
                send_sem=rs_send.at[i],
                recv_sem=rs_recv.at[i],
                device_id=tgt,
                device_id_type=pl.DeviceIdType.MESH,
            )
            d.start()
            rs.append(d)

        acc = x_ref[pl.ds(my_lid * rows, rows), :]
        for i, d in enumerate(rs):
            d.wait_recv()
            acc = acc + rs_buf[i, :, :]
        red_ref[:, :] = acc

        ag_sends = []
        ag_recvs = []
        for i, (tgt, tgt_lid) in enumerate(peers):
            send_d = pltpu.make_async_remote_copy(
                src_ref=red_ref,
                dst_ref=out_ref.at[pl.ds(my_lid * rows, rows)],
                send_sem=ag_send.at[i],
                recv_sem=ag_recv.at[i],
                device_id=tgt,
                device_id_type=pl.DeviceIdType.MESH,
            )
            send_d.start()
            ag_sends.append(send_d)
            recv_d = pltpu.make_async_remote_copy(
                src_ref=red_ref,
                dst_ref=out_ref.at[pl.ds(tgt_lid * rows, rows)],
                send_sem=ag_send.at[i],
                recv_sem=ag_recv.at[i],
                device_id=tgt,
                device_id_type=pl.DeviceIdType.MESH,
            )
            ag_recvs.append(recv_d)

        out_ref[pl.ds(my_lid * rows, rows), :] = red_ref[:, :]

        for d in ag_recvs:
            d.wait_recv()
        for d in rs:
            d.wait_send()
        for d in ag_sends:
            d.wait_send()

    return pl.pallas_call(
        body,
        out_shape=jax.ShapeDtypeStruct((m, n), x.dtype),
        in_specs=[pl.BlockSpec(memory_space=pltpu.VMEM)],
        out_specs=pl.BlockSpec(memory_space=pltpu.VMEM),
        scratch_shapes=[
            pltpu.VMEM((N_DEV - 1, rows, n), x.dtype),
            pltpu.VMEM((rows, n), x.dtype),
            pltpu.SemaphoreType.DMA((N_DEV - 1,)),
            pltpu.SemaphoreType.DMA((N_DEV - 1,)),
            pltpu.SemaphoreType.DMA((N_DEV - 1,)),
            pltpu.SemaphoreType.DMA((N_DEV - 1,)),
        ],
        compiler_params=pltpu.CompilerParams(collective_id=0),
    )(x)


# device time: 13344 ns/iter; 1.1131x vs baseline; 1.1131x over previous
import jax
import jax.numpy as jnp
from jax import lax
from jax.experimental import pallas as pl
from jax.experimental.pallas import tpu as pltpu

_CHUNKS = ((0, 88), (88, 88), (176, 80))
_ORDERS = ((2, 1, 0), (1, 0, 2), (0, 2, 1))


def kernel(x):
    m, n = x.shape[-2], x.shape[-1]
    x = x.reshape(m, n)

    def body(x_ref, out_ref, recv_buf, send_sems, recv_sems):
        my_x = lax.axis_index("x")
        my_y = lax.axis_index("y")
        my_z = lax.axis_index("z")
        nbr_by_axis = [
            (1 - my_x, my_y, my_z),
            (my_x, 1 - my_y, my_z),
            (my_x, my_y, 1 - my_z),
        ]

        barrier_sem = pltpu.get_barrier_semaphore()
        for nbr in nbr_by_axis:
            pl.semaphore_signal(
                barrier_sem, inc=1,
                device_id=nbr, device_id_type=pl.DeviceIdType.MESH,
            )
        pl.semaphore_wait(barrier_sem, 3)

        def exchange(p, c):
            src = x_ref if p == 0 else out_ref
            r0, rs = _CHUNKS[c]
            rdma = pltpu.make_async_remote_copy(
                src_ref=src.at[pl.ds(r0, rs)],
                dst_ref=recv_buf.at[p, pl.ds(r0, rs)],
                send_sem=send_sems.at[p, c],
                recv_sem=recv_sems.at[p, c],
                device_id=nbr_by_axis[_ORDERS[c][p]],
                device_id_type=pl.DeviceIdType.MESH,
            )
            rdma.start()
            return rdma

        rdmas = [exchange(0, c) for c in range(3)]
        for p in range(3):
            for c, (r0, rs) in enumerate(_CHUNKS):
                rdmas[c].wait()
                if p == 0:
                    out_ref[pl.ds(r0, rs), :] = (
                        x_ref[pl.ds(r0, rs), :] + recv_buf[0, pl.ds(r0, rs), :]
                    )
                else:
                    out_ref[pl.ds(r0, rs), :] += recv_buf[p, pl.ds(r0, rs), :]
                if p < 2:
                    rdmas[c] = exchange(p + 1, c)

    return pl.pallas_call(
        body,
        out_shape=jax.ShapeDtypeStruct((m, n), x.dtype),
        in_specs=[pl.BlockSpec(memory_space=pltpu.VMEM)],
        out_specs=pl.BlockSpec(memory_space=pltpu.VMEM),
        scratch_shapes=[
            pltpu.VMEM((3, m, n), x.dtype),
            pltpu.SemaphoreType.DMA((3, 3)),
            pltpu.SemaphoreType.DMA((3, 3)),
        ],
        compiler_params=pltpu.CompilerParams(collective_id=0),
    )(x)
